# Initial kernel scaffold; baseline (speedup 1.0000x reference)
import jax
import jax.numpy as jnp
from jax.experimental import pallas as pl


def kernel(x, adj, W, att_src, att_dst, bias):
    raise NotImplementedError("write your pallas kernel here")


# speedup vs baseline: 3.2947x; 3.2947x over previous
import functools

import jax
import jax.numpy as jnp
from jax.experimental import pallas as pl
from jax.experimental.pallas import tpu as pltpu

_ALPHA = 0.2


def _prep_kernel(x_ref, w_ref, attd_ref, h_ref, v_ref, q_ref):
    h = jnp.dot(x_ref[:], w_ref[:], preferred_element_type=jnp.float32)
    h_ref[:] = h
    ad = jax.lax.dot_general(attd_ref[:], h, (((1,), (1,)), ((), ())),
                             preferred_element_type=jnp.float32)
    v_ref[:] = jnp.exp(ad)
    q_ref[:] = jnp.exp(_ALPHA * ad)


def _gat_kernel(adj_ref, hs_ref, v_ref, q_ref, atts_ref, bias_ref,
                out_ref, denom_ref, *, bs, n, nsteps):
    i = pl.program_id(0)

    @pl.when(i == 0)
    def _init():
        out_ref[:] = jnp.zeros_like(out_ref)
        denom_ref[:] = jnp.zeros_like(denom_ref)

    hs = hs_ref[:]
    a_s = jax.lax.dot_general(hs, atts_ref[:], (((1,), (1,)), ((), ())),
                              preferred_element_type=jnp.float32)
    u = jnp.exp(a_s)
    p = jnp.exp(_ALPHA * a_s)

    rows = jax.lax.broadcasted_iota(jnp.int32, (bs, n), 0) + i * bs
    cols = jax.lax.broadcasted_iota(jnp.int32, (bs, n), 1)
    w = adj_ref[:] + jnp.where(rows == cols, 1.0, 0.0)
    P = w * jnp.maximum(u * v_ref[:], p * q_ref[:])

    out_ref[:] += jax.lax.dot_general(P, hs, (((0,), (0,)), ((), ())),
                                      preferred_element_type=jnp.float32)
    denom_ref[:] += jnp.sum(P, axis=0, keepdims=True)

    @pl.when(i == nsteps - 1)
    def _finalize():
        d = denom_ref[:].reshape(n, 1) + 1e-16
        o = out_ref[:] / d + bias_ref[:]
        out_ref[:] = jnp.where(o > 0, o, jnp.exp(o) - 1.0)


def _pick_bs(n):
    best = 8
    for cand in (512, 480, 400, 320, 256, 200, 160, 128, 80, 40, 16, 8):
        if n % cand == 0:
            return cand
    return best


def kernel(x, adj, W, att_src, att_dst, bias):
    n, nf = x.shape
    nh = W.shape[1]
    att_s2 = att_src.reshape(1, nh).astype(jnp.float32)
    att_d2 = att_dst.reshape(1, nh).astype(jnp.float32)
    bias2 = bias.reshape(1, nh).astype(jnp.float32)

    h, v, q = pl.pallas_call(
        _prep_kernel,
        out_shape=[
            jax.ShapeDtypeStruct((n, nh), jnp.float32),
            jax.ShapeDtypeStruct((1, n), jnp.float32),
            jax.ShapeDtypeStruct((1, n), jnp.float32),
        ],
    )(x, W, att_d2)

    bs = _pick_bs(n)
    nsteps = n // bs
    out = pl.pallas_call(
        functools.partial(_gat_kernel, bs=bs, n=n, nsteps=nsteps),
        grid=(nsteps,),
        in_specs=[
            pl.BlockSpec((bs, n), lambda i: (i, 0)),
            pl.BlockSpec((bs, nh), lambda i: (i, 0)),
            pl.BlockSpec((1, n), lambda i: (0, 0)),
            pl.BlockSpec((1, n), lambda i: (0, 0)),
            pl.BlockSpec((1, nh), lambda i: (0, 0)),
            pl.BlockSpec((1, nh), lambda i: (0, 0)),
        ],
        out_specs=pl.BlockSpec((n, nh), lambda i: (0, 0)),
        out_shape=jax.ShapeDtypeStruct((n, nh), jnp.float32),
        scratch_shapes=[pltpu.VMEM((1, n), jnp.float32)],
    )(adj, h, v, q, att_s2, bias2)
    return out


# speedup vs baseline: 3.5795x; 1.0865x over previous
import functools

import jax
import jax.numpy as jnp
from jax.experimental import pallas as pl
from jax.experimental.pallas import tpu as pltpu

_ALPHA = 0.2


def _prep_kernel(x_ref, w_ref, atts_ref, attd_ref,
                 h_ref, ht_ref, v_ref, q_ref, fd_ref):
    h = jnp.dot(x_ref[:], w_ref[:], preferred_element_type=jnp.float32)
    h_ref[:] = h
    ht_ref[:] = h.T
    ad = jax.lax.dot_general(attd_ref[:], h, (((1,), (1,)), ((), ())),
                             preferred_element_type=jnp.float32)
    a_s = jax.lax.dot_general(atts_ref[:], h, (((1,), (1,)), ((), ())),
                              preferred_element_type=jnp.float32)
    v_ref[:] = jnp.exp(ad)
    q_ref[:] = jnp.exp(_ALPHA * ad)
    z = a_s + ad
    fd_ref[:] = jnp.maximum(jnp.exp(z), jnp.exp(_ALPHA * z))


def _gat_kernel(adj_ref, hs_ref, v_ref, q_ref, atts_ref, ht_ref, fd_ref,
                bias_ref, out_ref, denom_ref, *, nsteps):
    i = pl.program_id(0)

    @pl.when(i == 0)
    def _init():
        out_ref[:] = jnp.zeros_like(out_ref)
        denom_ref[:] = jnp.zeros_like(denom_ref)

    hs = hs_ref[:]
    a_s = jax.lax.dot_general(hs, atts_ref[:], (((1,), (1,)), ((), ())),
                              preferred_element_type=jnp.float32)
    u = jnp.exp(a_s)
    p = jnp.exp(_ALPHA * a_s)

    P = adj_ref[:] * jnp.maximum(u * v_ref[:], p * q_ref[:])

    out_ref[:] += jax.lax.dot_general(hs, P, (((0,), (0,)), ((), ())),
                                      preferred_element_type=jnp.float32)
    denom_ref[:] += jnp.sum(P, axis=0, keepdims=True)

    @pl.when(i == nsteps - 1)
    def _finalize():
        fd = fd_ref[:]
        dn = denom_ref[:] + fd + 1e-16
        o = (out_ref[:] + fd * ht_ref[:]) / dn + bias_ref[:]
        out_ref[:] = jnp.where(o > 0, o, jnp.exp(o) - 1.0)


def _pick_bs(n):
    for cand in (256, 200, 160, 128, 80, 40, 16, 8):
        if n % cand == 0:
            return cand
    return 8


def kernel(x, adj, W, att_src, att_dst, bias):
    n, nf = x.shape
    nh = W.shape[1]
    att_s2 = att_src.reshape(1, nh).astype(jnp.float32)
    att_d2 = att_dst.reshape(1, nh).astype(jnp.float32)
    bias_t = bias.reshape(nh, 1).astype(jnp.float32)

    h, ht, v, q, fd = pl.pallas_call(
        _prep_kernel,
        out_shape=[
            jax.ShapeDtypeStruct((n, nh), jnp.float32),
            jax.ShapeDtypeStruct((nh, n), jnp.float32),
            jax.ShapeDtypeStruct((1, n), jnp.float32),
            jax.ShapeDtypeStruct((1, n), jnp.float32),
            jax.ShapeDtypeStruct((1, n), jnp.float32),
        ],
    )(x, W, att_s2, att_d2)

    bs = _pick_bs(n)
    nsteps = n // bs
    out_t = pl.pallas_call(
        functools.partial(_gat_kernel, nsteps=nsteps),
        grid=(nsteps,),
        in_specs=[
            pl.BlockSpec((bs, n), lambda i: (i, 0)),
            pl.BlockSpec((bs, nh), lambda i: (i, 0)),
            pl.BlockSpec((1, n), lambda i: (0, 0)),
            pl.BlockSpec((1, n), lambda i: (0, 0)),
            pl.BlockSpec((1, nh), lambda i: (0, 0)),
            pl.BlockSpec((nh, n), lambda i: (0, 0)),
            pl.BlockSpec((1, n), lambda i: (0, 0)),
            pl.BlockSpec((nh, 1), lambda i: (0, 0)),
        ],
        out_specs=pl.BlockSpec((nh, n), lambda i: (0, 0)),
        out_shape=jax.ShapeDtypeStruct((nh, n), jnp.float32),
        scratch_shapes=[pltpu.VMEM((1, n), jnp.float32)],
    )(adj, h, v, q, att_s2, ht, fd, bias_t)
    return out_t.T


# speedup vs baseline: 3.9529x; 1.1043x over previous
import functools

import jax
import jax.numpy as jnp
from jax.experimental import pallas as pl
from jax.experimental.pallas import tpu as pltpu

_ALPHA = 0.2


def _prep_kernel(x_ref, w_ref, atts_ref, attd_ref,
                 haug_ref, ht_ref, u_ref, p_ref, v_ref, q_ref, fd_ref):
    h = jnp.dot(x_ref[:], w_ref[:], preferred_element_type=jnp.float32)
    nh = h.shape[1]
    haug_ref[:, :nh] = h.astype(jnp.bfloat16)
    haug_ref[:, nh:] = jnp.ones_like(haug_ref[:, nh:])
    ht_ref[:] = h.T
    a_s_col = jax.lax.dot_general(h, atts_ref[:], (((1,), (1,)), ((), ())),
                                  preferred_element_type=jnp.float32)
    u_ref[:] = jnp.exp(a_s_col)
    p_ref[:] = jnp.exp(_ALPHA * a_s_col)
    ad = jax.lax.dot_general(attd_ref[:], h, (((1,), (1,)), ((), ())),
                             preferred_element_type=jnp.float32)
    a_s_row = jax.lax.dot_general(atts_ref[:], h, (((1,), (1,)), ((), ())),
                                  preferred_element_type=jnp.float32)
    v_ref[:] = jnp.exp(ad).astype(jnp.bfloat16)
    q_ref[:] = jnp.exp(_ALPHA * ad).astype(jnp.bfloat16)
    z = a_s_row + ad
    fd_ref[:] = jnp.maximum(jnp.exp(z), jnp.exp(_ALPHA * z))


def _gat_kernel(adj_ref, haug_ref, u_ref, p_ref, v_ref, q_ref,
                ht_ref, fd_ref, bias_ref, out_ref, *, nh, nsteps):
    i = pl.program_id(0)

    @pl.when(i == 0)
    def _init():
        out_ref[:] = jnp.zeros_like(out_ref)

    u = u_ref[:].astype(jnp.bfloat16)
    p = p_ref[:].astype(jnp.bfloat16)
    m = jnp.maximum(u * v_ref[:], p * q_ref[:])
    P = adj_ref[:].astype(jnp.bfloat16) * m

    out_ref[:] += jax.lax.dot_general(haug_ref[:], P, (((0,), (0,)), ((), ())),
                                      preferred_element_type=jnp.float32)

    @pl.when(i == nsteps - 1)
    def _finalize():
        fd = fd_ref[:]
        dn = out_ref[nh:nh + 1, :] + fd + 1e-16
        o = (out_ref[:nh, :] + fd * ht_ref[:]) / dn + bias_ref[:]
        out_ref[:nh, :] = jnp.where(o > 0, o, jnp.exp(o) - 1.0)


def _pick_bs(n):
    for cand in (400, 320, 256, 200, 160, 128, 80, 40, 16, 8):
        if n % cand == 0:
            return cand
    return 8


def kernel(x, adj, W, att_src, att_dst, bias):
    n, nf = x.shape
    nh = W.shape[1]
    att_s2 = att_src.reshape(1, nh).astype(jnp.float32)
    att_d2 = att_dst.reshape(1, nh).astype(jnp.float32)
    bias_t = bias.reshape(nh, 1).astype(jnp.float32)

    haug, ht, u, p, v, q, fd = pl.pallas_call(
        _prep_kernel,
        out_shape=[
            jax.ShapeDtypeStruct((n, nh + 8), jnp.bfloat16),
            jax.ShapeDtypeStruct((nh, n), jnp.float32),
            jax.ShapeDtypeStruct((n, 1), jnp.float32),
            jax.ShapeDtypeStruct((n, 1), jnp.float32),
            jax.ShapeDtypeStruct((1, n), jnp.bfloat16),
            jax.ShapeDtypeStruct((1, n), jnp.bfloat16),
            jax.ShapeDtypeStruct((1, n), jnp.float32),
        ],
    )(x, W, att_s2, att_d2)

    bs = _pick_bs(n)
    nsteps = n // bs
    out_t = pl.pallas_call(
        functools.partial(_gat_kernel, nh=nh, nsteps=nsteps),
        grid=(nsteps,),
        in_specs=[
            pl.BlockSpec((bs, n), lambda i: (i, 0)),
            pl.BlockSpec((bs, nh + 8), lambda i: (i, 0)),
            pl.BlockSpec((bs, 1), lambda i: (i, 0)),
            pl.BlockSpec((bs, 1), lambda i: (i, 0)),
            pl.BlockSpec((1, n), lambda i: (0, 0)),
            pl.BlockSpec((1, n), lambda i: (0, 0)),
            pl.BlockSpec((nh, n), lambda i: (0, 0)),
            pl.BlockSpec((1, n), lambda i: (0, 0)),
            pl.BlockSpec((nh, 1), lambda i: (0, 0)),
        ],
        out_specs=pl.BlockSpec((nh + 8, n), lambda i: (0, 0)),
        out_shape=jax.ShapeDtypeStruct((nh + 8, n), jnp.float32),
    )(adj, haug, u, p, v, q, ht, fd, bias_t)
    return out_t[:nh].T


# speedup vs baseline: 4.1741x; 1.0560x over previous
import functools

import jax
import jax.numpy as jnp
from jax.experimental import pallas as pl
from jax.experimental.pallas import tpu as pltpu

_ALPHA = 0.2


def _prep_kernel(x_ref, w_ref, atts_ref, attd_ref,
                 haug_ref, ht_ref, u_ref, p_ref, v_ref, q_ref, fd_ref):
    h = jnp.dot(x_ref[:], w_ref[:], preferred_element_type=jnp.float32)
    nh = h.shape[1]
    haug_ref[:, :nh] = h.astype(jnp.bfloat16)
    haug_ref[:, nh:] = jnp.ones_like(haug_ref[:, nh:])
    ht_ref[:] = h.T
    a_s_col = jax.lax.dot_general(h, atts_ref[:], (((1,), (1,)), ((), ())),
                                  preferred_element_type=jnp.float32)
    u_ref[:] = jnp.exp(a_s_col)
    p_ref[:] = jnp.exp(_ALPHA * a_s_col)
    ad = jax.lax.dot_general(attd_ref[:], h, (((1,), (1,)), ((), ())),
                             preferred_element_type=jnp.float32)
    a_s_row = jax.lax.dot_general(atts_ref[:], h, (((1,), (1,)), ((), ())),
                                  preferred_element_type=jnp.float32)
    v_ref[:] = jnp.exp(ad).astype(jnp.bfloat16)
    q_ref[:] = jnp.exp(_ALPHA * ad).astype(jnp.bfloat16)
    z = a_s_row + ad
    fd_ref[:] = jnp.maximum(jnp.exp(z), jnp.exp(_ALPHA * z))


def _gat_kernel(adj_ref, haug_ref, u_ref, p_ref, v_ref, q_ref,
                ht_ref, fd_ref, bias_ref, out_ref, acc_ref, *, nh, nsteps):
    i = pl.program_id(0)

    @pl.when(i == 0)
    def _init():
        acc_ref[:] = jnp.zeros_like(acc_ref)

    u = u_ref[:].astype(jnp.bfloat16)
    p = p_ref[:].astype(jnp.bfloat16)
    m = jnp.maximum(u * v_ref[:], p * q_ref[:])
    P = adj_ref[:].astype(jnp.bfloat16) * m

    acc_ref[:] += jax.lax.dot_general(haug_ref[:], P, (((0,), (0,)), ((), ())),
                                      preferred_element_type=jnp.float32)

    @pl.when(i == nsteps - 1)
    def _finalize():
        fd = fd_ref[:]
        dn = acc_ref[nh:nh + 1, :] + fd + 1e-16
        o = (acc_ref[:nh, :] + fd * ht_ref[:]) / dn + bias_ref[:]
        out_ref[:] = jnp.where(o > 0, o, jnp.exp(o) - 1.0).T


def _pick_bs(n):
    for cand in (400, 320, 256, 200, 160, 128, 80, 40, 16, 8):
        if n % cand == 0:
            return cand
    return 8


def kernel(x, adj, W, att_src, att_dst, bias):
    n, nf = x.shape
    nh = W.shape[1]
    att_s2 = att_src.reshape(1, nh).astype(jnp.float32)
    att_d2 = att_dst.reshape(1, nh).astype(jnp.float32)
    bias_t = bias.reshape(nh, 1).astype(jnp.float32)

    haug, ht, u, p, v, q, fd = pl.pallas_call(
        _prep_kernel,
        out_shape=[
            jax.ShapeDtypeStruct((n, nh + 8), jnp.bfloat16),
            jax.ShapeDtypeStruct((nh, n), jnp.float32),
            jax.ShapeDtypeStruct((n, 1), jnp.float32),
            jax.ShapeDtypeStruct((n, 1), jnp.float32),
            jax.ShapeDtypeStruct((1, n), jnp.bfloat16),
            jax.ShapeDtypeStruct((1, n), jnp.bfloat16),
            jax.ShapeDtypeStruct((1, n), jnp.float32),
        ],
    )(x, W, att_s2, att_d2)

    bs = _pick_bs(n)
    nsteps = n // bs
    out_t = pl.pallas_call(
        functools.partial(_gat_kernel, nh=nh, nsteps=nsteps),
        grid=(nsteps,),
        in_specs=[
            pl.BlockSpec((bs, n), lambda i: (i, 0)),
            pl.BlockSpec((bs, nh + 8), lambda i: (i, 0)),
            pl.BlockSpec((bs, 1), lambda i: (i, 0)),
            pl.BlockSpec((bs, 1), lambda i: (i, 0)),
            pl.BlockSpec((1, n), lambda i: (0, 0)),
            pl.BlockSpec((1, n), lambda i: (0, 0)),
            pl.BlockSpec((nh, n), lambda i: (0, 0)),
            pl.BlockSpec((1, n), lambda i: (0, 0)),
            pl.BlockSpec((nh, 1), lambda i: (0, 0)),
        ],
        out_specs=pl.BlockSpec((n, nh), lambda i: (0, 0)),
        out_shape=jax.ShapeDtypeStruct((n, nh), jnp.float32),
        scratch_shapes=[pltpu.VMEM((nh + 8, n), jnp.float32)],
    )(adj, haug, u, p, v, q, ht, fd, bias_t)
    return out_t
